# baseline (speedup 1.0000x reference)
import jax
import jax.numpy as jnp
from jax.experimental import pallas as pl

N_PAIRS = 1000
NODES_PER_G = 5
EDGES_PER_G = 80
MAX_EDGES = 96
STATE = 64
MSG = 128
TDIM = 16
N_PROP = 2
SINKHORN_ITERS = 3
TEMP = 0.1

PB = 8
GRID = N_PAIRS // PB
NG = 2 * PB
PAD = MAX_EDGES - EDGES_PER_G

_INTERPRET = False
_LO = jax.lax.Precision.DEFAULT


def _dot(a, b):
    return jax.lax.dot_general(a, b, (((a.ndim - 1,), (0,)), ((), ())),
                               precision=_LO,
                               preferred_element_type=jnp.float32)


def _bdot(a, b, ca, cb):
    return jax.lax.dot_general(a, b, (((ca,), (cb,)), ((0,), (0,))),
                               precision=_LO,
                               preferred_element_type=jnp.float32)


def _isonet_block(nf_ref, ef_ref, fr_ref, to_ref,
                  wen_ref, ben_ref, wee_ref, bee_ref,
                  wm_ref, bm_ref, wr_ref, br_ref,
                  wu_ref, bu_ref, w1_ref, b1_ref, w2_ref, b2_ref,
                  out_ref):
    lf = fr_ref[0]
    lt = to_ref[0]
    lfg = lf - NODES_PER_G * (lf // NODES_PER_G)
    ltg = lt - NODES_PER_G * (lt // NODES_PER_G)
    five = jax.lax.broadcasted_iota(jnp.int32, (NG, EDGES_PER_G, NODES_PER_G), 2)
    F5 = (lfg == five).astype(jnp.float32)
    T5 = (ltg == five).astype(jnp.float32)

    h = _dot(nf_ref[0], wen_ref[...]) + ben_ref[...]
    e = _dot(ef_ref[0], wee_ref[...]) + bee_ref[...]

    wm = wm_ref[...]
    wr = wr_ref[...]
    Wm1, Wm2, Wm3 = wm[0:STATE], wm[STATE:2 * STATE], wm[2 * STATE:]
    Wr1, Wr2, Wr3 = wr[0:STATE], wr[STATE:2 * STATE], wr[2 * STATE:]
    bm = bm_ref[...]
    br = br_ref[...]

    C = _bdot(F5, T5, 1, 1)
    ones5 = jnp.ones((NODES_PER_G, 1), jnp.float32)
    indeg = _dot(C.swapaxes(1, 2), ones5)
    outdeg = _dot(C, ones5)
    S = (_dot(_bdot(T5, e, 1, 1), Wm3) + _dot(_bdot(F5, e, 1, 1), Wr3)
         + indeg * bm + outdeg * br)

    wu = wu_ref[...]
    Wu1, Wu2 = wu[0:STATE], wu[STATE:]
    bu = bu_ref[...]
    for _ in range(N_PROP):
        hin = _bdot(C, h, 1, 1)
        hout = _bdot(C, h, 2, 1)
        agg = (_dot(hin, Wm1) + _dot(hout, Wr1)
               + indeg * _dot(h, Wm2) + outdeg * _dot(h, Wr2) + S)
        h = _dot(h, Wu1) + _dot(agg, Wu2) + bu

    Pn = _dot(h, Wm1) + _dot(h, Wr2)
    Qn = _dot(h, Wm2) + _dot(h, Wr1)
    eenc = _dot(e, Wm3) + _dot(e, Wr3) + bm + br
    for r in range(NODES_PER_G):
        eenc = eenc + jnp.where(lfg == r, Pn[:, r:r + 1, :], 0.0)
        eenc = eenc + jnp.where(ltg == r, Qn[:, r:r + 1, :], 0.0)

    t = _dot(jax.nn.relu(_dot(eenc, w1_ref[...]) + b1_ref[...]),
             w2_ref[...]) + b2_ref[...]

    e4 = eenc.reshape(PB, 2, EDGES_PER_G, MSG)
    q = jnp.concatenate([e4[:, 0], jnp.zeros((PB, PAD, MSG), jnp.float32)], axis=1)
    c = jnp.concatenate([e4[:, 1], jnp.zeros((PB, PAD, MSG), jnp.float32)], axis=1)
    t4 = t.reshape(PB, 2, EDGES_PER_G, TDIM)
    mq = jnp.concatenate([t4[:, 0], jnp.zeros((PB, PAD, TDIM), jnp.float32)], axis=1)
    mc = jnp.concatenate([t4[:, 1], jnp.zeros((PB, PAD, TDIM), jnp.float32)], axis=1)

    la = jax.lax.dot_general(mq, mc, (((2,), (2,)), ((0,), (0,))),
                             precision=_LO,
                             preferred_element_type=jnp.float32) / TEMP
    for _ in range(SINKHORN_ITERS):
        m2 = jnp.max(la, axis=2, keepdims=True)
        la = la - (m2 + jnp.log(jnp.sum(jnp.exp(la - m2), axis=2, keepdims=True)))
        m1 = jnp.max(la, axis=1, keepdims=True)
        la = la - (m1 + jnp.log(jnp.sum(jnp.exp(la - m1), axis=1, keepdims=True)))
    plan = jnp.exp(la)
    r2 = jax.lax.dot_general(plan, c, (((2,), (1,)), ((0,), (0,))),
                             precision=_LO,
                             preferred_element_type=jnp.float32)
    d = jax.nn.relu(q - r2)
    out_ref[0] = -jnp.sum(jnp.sum(d, axis=2), axis=1, keepdims=True)


def kernel(node_features, edge_features, from_idx, to_idx, graph_idx,
           W_enc_n, b_enc_n, W_enc_e, b_enc_e, W_msg, b_msg, W_rmsg, b_rmsg,
           W_upd, b_upd, W_fc1, b_fc1, W_fc2, b_fc2):
    del graph_idx
    nf = node_features.reshape(GRID, NG, NODES_PER_G, node_features.shape[1])
    ef = edge_features.reshape(GRID, NG, EDGES_PER_G, edge_features.shape[1])
    fr = from_idx.reshape(GRID, NG, EDGES_PER_G, 1)
    to = to_idx.reshape(GRID, NG, EDGES_PER_G, 1)

    def row(v):
        return v.reshape(1, -1)

    def full(shape):
        return pl.BlockSpec(shape, lambda i: (0,) * len(shape))

    out = pl.pallas_call(
        _isonet_block,
        grid=(GRID,),
        in_specs=[
            pl.BlockSpec((1,) + nf.shape[1:], lambda i: (i, 0, 0, 0)),
            pl.BlockSpec((1,) + ef.shape[1:], lambda i: (i, 0, 0, 0)),
            pl.BlockSpec((1, NG, EDGES_PER_G, 1), lambda i: (i, 0, 0, 0)),
            pl.BlockSpec((1, NG, EDGES_PER_G, 1), lambda i: (i, 0, 0, 0)),
            full(W_enc_n.shape), full((1, STATE)),
            full(W_enc_e.shape), full((1, STATE)),
            full(W_msg.shape), full((1, MSG)),
            full(W_rmsg.shape), full((1, MSG)),
            full(W_upd.shape), full((1, STATE)),
            full(W_fc1.shape), full((1, TDIM)),
            full(W_fc2.shape), full((1, TDIM)),
        ],
        out_specs=pl.BlockSpec((1, PB, 1), lambda i: (i, 0, 0)),
        out_shape=jax.ShapeDtypeStruct((GRID, PB, 1), jnp.float32),
        interpret=_INTERPRET,
    )(nf, ef, fr, to,
      W_enc_n, row(b_enc_n), W_enc_e, row(b_enc_e),
      W_msg, row(b_msg), W_rmsg, row(b_rmsg),
      W_upd, row(b_upd), W_fc1, row(b_fc1), W_fc2, row(b_fc2))
    return out.reshape(N_PAIRS)


# speedup vs baseline: 1.1111x; 1.1111x over previous
import jax
import jax.numpy as jnp
from jax.experimental import pallas as pl

N_PAIRS = 1000
NODES_PER_G = 5
EDGES_PER_G = 80
MAX_EDGES = 96
STATE = 64
MSG = 128
TDIM = 16
N_PROP = 2
SINKHORN_ITERS = 3
TEMP = 0.1

PB = 40
GRID = N_PAIRS // PB
NG = 2 * PB
PAD = MAX_EDGES - EDGES_PER_G

_INTERPRET = False
_LO = jax.lax.Precision.DEFAULT


def _dot(a, b):
    return jax.lax.dot_general(a, b, (((a.ndim - 1,), (0,)), ((), ())),
                               precision=_LO,
                               preferred_element_type=jnp.float32)


def _bdot(a, b, ca, cb):
    return jax.lax.dot_general(a, b, (((ca,), (cb,)), ((0,), (0,))),
                               precision=_LO,
                               preferred_element_type=jnp.float32)


def _isonet_block(nf_ref, ef_ref, fr_ref, to_ref,
                  wen_ref, ben_ref, wee_ref, bee_ref,
                  wm_ref, bm_ref, wr_ref, br_ref,
                  wu_ref, bu_ref, w1_ref, b1_ref, w2_ref, b2_ref,
                  out_ref):
    lf = fr_ref[0]
    lt = to_ref[0]
    lfg = lf - NODES_PER_G * (lf // NODES_PER_G)
    ltg = lt - NODES_PER_G * (lt // NODES_PER_G)
    five = jax.lax.broadcasted_iota(jnp.int32, (NG, EDGES_PER_G, NODES_PER_G), 2)
    F5 = (lfg == five).astype(jnp.float32)
    T5 = (ltg == five).astype(jnp.float32)

    h = _dot(nf_ref[0], wen_ref[...]) + ben_ref[...]
    e = _dot(ef_ref[0], wee_ref[...]) + bee_ref[...]

    wm = wm_ref[...]
    wr = wr_ref[...]
    Wm1, Wm2, Wm3 = wm[0:STATE], wm[STATE:2 * STATE], wm[2 * STATE:]
    Wr1, Wr2, Wr3 = wr[0:STATE], wr[STATE:2 * STATE], wr[2 * STATE:]
    bm = bm_ref[...]
    br = br_ref[...]

    C = _bdot(F5, T5, 1, 1)
    ones5 = jnp.ones((NODES_PER_G, 1), jnp.float32)
    indeg = _dot(C.swapaxes(1, 2), ones5)
    outdeg = _dot(C, ones5)
    S = (_dot(_bdot(T5, e, 1, 1), Wm3) + _dot(_bdot(F5, e, 1, 1), Wr3)
         + indeg * bm + outdeg * br)

    wu = wu_ref[...]
    Wu1, Wu2 = wu[0:STATE], wu[STATE:]
    bu = bu_ref[...]
    for _ in range(N_PROP):
        hin = _bdot(C, h, 1, 1)
        hout = _bdot(C, h, 2, 1)
        agg = (_dot(hin, Wm1) + _dot(hout, Wr1)
               + indeg * _dot(h, Wm2) + outdeg * _dot(h, Wr2) + S)
        h = _dot(h, Wu1) + _dot(agg, Wu2) + bu

    Pn = _dot(h, Wm1) + _dot(h, Wr2)
    Qn = _dot(h, Wm2) + _dot(h, Wr1)
    eenc = _dot(e, Wm3) + _dot(e, Wr3) + bm + br
    for r in range(NODES_PER_G):
        eenc = eenc + jnp.where(lfg == r, Pn[:, r:r + 1, :], 0.0)
        eenc = eenc + jnp.where(ltg == r, Qn[:, r:r + 1, :], 0.0)

    t = _dot(jax.nn.relu(_dot(eenc, w1_ref[...]) + b1_ref[...]),
             w2_ref[...]) + b2_ref[...]

    e4 = eenc.reshape(PB, 2, EDGES_PER_G, MSG)
    q = jnp.concatenate([e4[:, 0], jnp.zeros((PB, PAD, MSG), jnp.float32)], axis=1)
    c = jnp.concatenate([e4[:, 1], jnp.zeros((PB, PAD, MSG), jnp.float32)], axis=1)
    t4 = t.reshape(PB, 2, EDGES_PER_G, TDIM)
    mq = jnp.concatenate([t4[:, 0], jnp.zeros((PB, PAD, TDIM), jnp.float32)], axis=1)
    mc = jnp.concatenate([t4[:, 1], jnp.zeros((PB, PAD, TDIM), jnp.float32)], axis=1)

    la = jax.lax.dot_general(mq, mc, (((2,), (2,)), ((0,), (0,))),
                             precision=_LO,
                             preferred_element_type=jnp.float32) / TEMP
    for _ in range(SINKHORN_ITERS):
        m2 = jnp.max(la, axis=2, keepdims=True)
        la = la - (m2 + jnp.log(jnp.sum(jnp.exp(la - m2), axis=2, keepdims=True)))
        m1 = jnp.max(la, axis=1, keepdims=True)
        la = la - (m1 + jnp.log(jnp.sum(jnp.exp(la - m1), axis=1, keepdims=True)))
    plan = jnp.exp(la)
    r2 = jax.lax.dot_general(plan, c, (((2,), (1,)), ((0,), (0,))),
                             precision=_LO,
                             preferred_element_type=jnp.float32)
    d = jax.nn.relu(q - r2)
    out_ref[0] = -jnp.sum(jnp.sum(d, axis=2), axis=1, keepdims=True)


def kernel(node_features, edge_features, from_idx, to_idx, graph_idx,
           W_enc_n, b_enc_n, W_enc_e, b_enc_e, W_msg, b_msg, W_rmsg, b_rmsg,
           W_upd, b_upd, W_fc1, b_fc1, W_fc2, b_fc2):
    del graph_idx
    nf = node_features.reshape(GRID, NG, NODES_PER_G, node_features.shape[1])
    ef = edge_features.reshape(GRID, NG, EDGES_PER_G, edge_features.shape[1])
    fr = from_idx.reshape(GRID, NG, EDGES_PER_G, 1)
    to = to_idx.reshape(GRID, NG, EDGES_PER_G, 1)

    def row(v):
        return v.reshape(1, -1)

    def full(shape):
        return pl.BlockSpec(shape, lambda i: (0,) * len(shape))

    out = pl.pallas_call(
        _isonet_block,
        grid=(GRID,),
        in_specs=[
            pl.BlockSpec((1,) + nf.shape[1:], lambda i: (i, 0, 0, 0)),
            pl.BlockSpec((1,) + ef.shape[1:], lambda i: (i, 0, 0, 0)),
            pl.BlockSpec((1, NG, EDGES_PER_G, 1), lambda i: (i, 0, 0, 0)),
            pl.BlockSpec((1, NG, EDGES_PER_G, 1), lambda i: (i, 0, 0, 0)),
            full(W_enc_n.shape), full((1, STATE)),
            full(W_enc_e.shape), full((1, STATE)),
            full(W_msg.shape), full((1, MSG)),
            full(W_rmsg.shape), full((1, MSG)),
            full(W_upd.shape), full((1, STATE)),
            full(W_fc1.shape), full((1, TDIM)),
            full(W_fc2.shape), full((1, TDIM)),
        ],
        out_specs=pl.BlockSpec((1, PB, 1), lambda i: (i, 0, 0)),
        out_shape=jax.ShapeDtypeStruct((GRID, PB, 1), jnp.float32),
        interpret=_INTERPRET,
    )(nf, ef, fr, to,
      W_enc_n, row(b_enc_n), W_enc_e, row(b_enc_e),
      W_msg, row(b_msg), W_rmsg, row(b_rmsg),
      W_upd, row(b_upd), W_fc1, row(b_fc1), W_fc2, row(b_fc2))
    return out.reshape(N_PAIRS)
